# Initial kernel scaffold; baseline (speedup 1.0000x reference)
import jax
import jax.numpy as jnp
from jax.experimental import pallas as pl


def kernel(x, W):
    raise NotImplementedError("write your pallas kernel here")


# speedup vs baseline: 1.7960x; 1.7960x over previous
import functools

import jax
import jax.numpy as jnp
from jax.experimental import pallas as pl

_TOPK = 8
_NEXP = 64


def _gate_block(x_ref, w_ref, wout_ref, iout_ref):
    x = x_ref[...]
    w = w_ref[...]
    scores = jax.lax.dot_general(
        x, w, (((1,), (1,)), ((), ())), preferred_element_type=jnp.float32
    )
    scores = jax.nn.sigmoid(scores)
    rows = scores.shape[0]
    iota = jax.lax.broadcasted_iota(jnp.int32, (rows, _NEXP), 1)
    work = scores
    vals = []
    idxs = []
    for _ in range(_TOPK):
        m = jnp.max(work, axis=1, keepdims=True)
        cand = jnp.where(work == m, iota, _NEXP)
        idx = jnp.min(cand, axis=1, keepdims=True)
        vals.append(m)
        idxs.append(idx)
        work = jnp.where(iota == idx, -jnp.inf, work)
    total = vals[0]
    for v in vals[1:]:
        total = total + v
    wout_ref[...] = jnp.concatenate(vals, axis=1) / total
    iout_ref[...] = jnp.concatenate(idxs, axis=1)


@jax.jit
def kernel(x, W):
    tokens = x.shape[0]
    block = 1024
    grid = tokens // block
    wout, iout = pl.pallas_call(
        _gate_block,
        grid=(grid,),
        in_specs=[
            pl.BlockSpec((block, x.shape[1]), lambda i: (i, 0)),
            pl.BlockSpec((_NEXP, x.shape[1]), lambda i: (0, 0)),
        ],
        out_specs=[
            pl.BlockSpec((block, _TOPK), lambda i: (i, 0)),
            pl.BlockSpec((block, _TOPK), lambda i: (i, 0)),
        ],
        out_shape=[
            jax.ShapeDtypeStruct((tokens, _TOPK), jnp.float32),
            jax.ShapeDtypeStruct((tokens, _TOPK), jnp.int32),
        ],
    )(x, W)
    return (wout, iout)
